# Initial kernel scaffold; baseline (speedup 1.0000x reference)
import functools

import jax
import jax.numpy as jnp
from jax import lax
from jax.experimental import pallas as pl
from jax.experimental.pallas import tpu as pltpu
from jax.experimental.pallas import tpu_sc as plsc

EMB = 16
N_NODES = 100000
N_EDGES = 3200000
NC = 2
NS = 16
NW = NC * NS
CH = 128
N_CHUNKS = N_EDGES // CH
BASE_CHUNKS = N_CHUNKS // NW
EXTRA = N_CHUNKS - BASE_CHUNKS * NW

ROWS_PER_TILE = N_NODES // NS
ZROW = 125
CNT_CH = 800
CNT_NCH = N_NODES // CNT_CH


def _sc_edge_kernel(L_hbm, R_hbm, lidx_hbm, ridx_hbm, w_hbm, we_hbm,
                    acc_out, cnt_out,
                    lidx_v, ridx_v, wch_v, lrows, rrows, mrows,
                    we_v, ones_v, zrow_v, acc_sh, cnt_sh, sem1, sem2):
  cid = lax.axis_index("c")
  sid = lax.axis_index("s")
  wid = sid * NC + cid

  zero16 = jnp.zeros((16,), jnp.float32)

  def _z_m(i, _):
    mrows[i] = zero16
    return 0
  lax.fori_loop(0, CH, _z_m, 0)

  def _z_zrow(i, _):
    zrow_v[pl.ds(i * 16, 16)] = zero16
    return 0
  lax.fori_loop(0, CNT_CH // 16, _z_zrow, 0)

  def _o_ones(i, _):
    ones_v[pl.ds(i * 16, 16)] = jnp.ones((16,), jnp.float32)
    return 0
  lax.fori_loop(0, CH // 16, _o_ones, 0)

  pltpu.sync_copy(we_hbm, we_v)

  def _z_acc(k, _):
    pltpu.sync_copy(mrows.at[pl.ds(0, ZROW)],
                    acc_sh.at[pl.ds(sid * ROWS_PER_TILE + k * ZROW, ZROW)])
    return 0
  lax.fori_loop(0, ROWS_PER_TILE // ZROW, _z_acc, 0)

  def _z_cnt(k2, _):
    k = sid + k2 * NS

    @pl.when(k < CNT_NCH)
    def _():
      pltpu.sync_copy(zrow_v.at[pl.ds(0, CNT_CH)],
                      cnt_sh.at[pl.ds(k * CNT_CH, CNT_CH)])
    return 0
  lax.fori_loop(0, (CNT_NCH + NS - 1) // NS, _z_cnt, 0)

  plsc.subcore_barrier()

  base = wid * BASE_CHUNKS + jnp.minimum(wid, EXTRA)
  count = BASE_CHUNKS + jnp.where(wid < EXTRA, 1, 0)

  def _chunk(k, _):
    off = (base + k) * CH
    pltpu.sync_copy(lidx_hbm.at[pl.ds(off, CH)], lidx_v)
    pltpu.sync_copy(ridx_hbm.at[pl.ds(off, CH)], ridx_v)
    pltpu.sync_copy(w_hbm.at[pl.ds(off, CH)], wch_v)
    cl = pltpu.async_copy(L_hbm.at[lidx_v], lrows, sem1)
    cr = pltpu.async_copy(R_hbm.at[ridx_v], rrows, sem2)
    cl.wait()
    cr.wait()

    we = we_v[...]

    def _edge(i, _):
      wv = plsc.load_gather(wch_v, [jnp.full((16,), i, jnp.int32)])
      m = jnp.maximum(lrows[i] + rrows[i] + wv * we, 0.0)
      mrows[i] = m
      return 0
    lax.fori_loop(0, CH, _edge, 0)

    pltpu.sync_copy(mrows, acc_sh.at[ridx_v], add=True)
    pltpu.sync_copy(ones_v, cnt_sh.at[ridx_v], add=True)
    return 0
  lax.fori_loop(0, count, _chunk, 0)

  plsc.subcore_barrier()

  def _cp_acc(k, _):
    r0 = sid * ROWS_PER_TILE + k * ZROW
    pltpu.sync_copy(acc_sh.at[pl.ds(r0, ZROW)],
                    acc_out.at[cid, pl.ds(r0, ZROW)])
    return 0
  lax.fori_loop(0, ROWS_PER_TILE // ZROW, _cp_acc, 0)

  def _cp_cnt(k2, _):
    k = sid + k2 * NS

    @pl.when(k < CNT_NCH)
    def _():
      pltpu.sync_copy(cnt_sh.at[pl.ds(k * CNT_CH, CNT_CH)],
                      cnt_out.at[cid, pl.ds(k * CNT_CH, CNT_CH)])
    return 0
  lax.fori_loop(0, (CNT_NCH + NS - 1) // NS, _cp_cnt, 0)


def _sc_edge(L, R, lidx, ridx, w, we):
  mesh = plsc.VectorSubcoreMesh(core_axis_name="c", subcore_axis_name="s")
  f = pl.kernel(
      _sc_edge_kernel,
      out_type=(
          jax.ShapeDtypeStruct((NC, N_NODES, EMB), jnp.float32),
          jax.ShapeDtypeStruct((NC, N_NODES), jnp.float32),
      ),
      mesh=mesh,
      scratch_types=[
          pltpu.VMEM((CH,), jnp.int32),
          pltpu.VMEM((CH,), jnp.int32),
          pltpu.VMEM((CH,), jnp.float32),
          pltpu.VMEM((CH, EMB), jnp.float32),
          pltpu.VMEM((CH, EMB), jnp.float32),
          pltpu.VMEM((CH, EMB), jnp.float32),
          pltpu.VMEM((EMB,), jnp.float32),
          pltpu.VMEM((CH,), jnp.float32),
          pltpu.VMEM((CNT_CH,), jnp.float32),
          pltpu.VMEM_SHARED((N_NODES, EMB), jnp.float32),
          pltpu.VMEM_SHARED((N_NODES,), jnp.float32),
          pltpu.SemaphoreType.DMA,
          pltpu.SemaphoreType.DMA,
      ],
  )
  return f(L, R, lidx, ridx, w, we)


_BLK = 2000
_GRID = N_NODES // _BLK


def _pre_kernel(lf_ref, rf_ref, wl_ref, bl_ref, wr_ref, L_ref, R_ref):
  L_ref[...] = jnp.dot(lf_ref[...], wl_ref[...],
                       preferred_element_type=jnp.float32) + bl_ref[...]
  R_ref[...] = jnp.dot(rf_ref[...], wr_ref[...],
                       preferred_element_type=jnp.float32)


def _tc_pre(lf, rf, W_l, b_l, W_r):
  row_spec = pl.BlockSpec((_BLK, EMB), lambda i: (i, 0))
  w_spec = pl.BlockSpec((EMB, EMB), lambda i: (0, 0))
  b_spec = pl.BlockSpec((1, EMB), lambda i: (0, 0))
  return pl.pallas_call(
      _pre_kernel,
      grid=(_GRID,),
      in_specs=[row_spec, row_spec, w_spec, b_spec, w_spec],
      out_specs=[row_spec, row_spec],
      out_shape=[
          jax.ShapeDtypeStruct((N_NODES, EMB), jnp.float32),
          jax.ShapeDtypeStruct((N_NODES, EMB), jnp.float32),
      ],
  )(lf, rf, W_l, b_l.reshape(1, EMB), W_r)


def _post_kernel(acc0_ref, acc1_ref, cnt_ref, rf_ref, wf_ref, bf_ref,
                 wp_ref, bp_ref, wo1a_ref, wo1b_ref, bo1_ref, wo2_ref,
                 bo2_ref, out_ref):
  acc = acc0_ref[...] + acc1_ref[...]
  n = jnp.sum(cnt_ref[...], axis=1, keepdims=True)
  agg = jnp.dot(acc, wf_ref[...],
                preferred_element_type=jnp.float32) + n * bf_ref[...]
  post = jnp.dot(jnp.maximum(agg, 0.0), wp_ref[...],
                 preferred_element_type=jnp.float32) + bp_ref[...]
  h = jnp.maximum(
      jnp.dot(post, wo1a_ref[...], preferred_element_type=jnp.float32)
      + jnp.dot(rf_ref[...], wo1b_ref[...],
                preferred_element_type=jnp.float32)
      + bo1_ref[...], 0.0)
  out_ref[...] = jnp.dot(h, wo2_ref[...],
                         preferred_element_type=jnp.float32) + bo2_ref[...]


def _tc_post(acc0, acc1, cntT, rf, W_f, b_f, W_p, b_p, W_o1a, W_o1b, b_o1,
             W_o2, b_o2):
  row_spec = pl.BlockSpec((_BLK, EMB), lambda i: (i, 0))
  cnt_spec = pl.BlockSpec((_BLK, NC), lambda i: (i, 0))
  w_spec = pl.BlockSpec((EMB, EMB), lambda i: (0, 0))
  b_spec = pl.BlockSpec((1, EMB), lambda i: (0, 0))
  return pl.pallas_call(
      _post_kernel,
      grid=(_GRID,),
      in_specs=[row_spec, row_spec, cnt_spec, row_spec,
                w_spec, b_spec, w_spec, b_spec,
                w_spec, w_spec, b_spec, w_spec, b_spec],
      out_specs=row_spec,
      out_shape=jax.ShapeDtypeStruct((N_NODES, EMB), jnp.float32),
  )(acc0, acc1, cntT, rf, W_f, b_f.reshape(1, EMB), W_p,
    b_p.reshape(1, EMB), W_o1a, W_o1b, b_o1.reshape(1, EMB), W_o2,
    b_o2.reshape(1, EMB))


@jax.jit
def kernel(left_features, edge_indices, edge_features, right_features,
           W_l, b_l, W_e, W_r, W_f, b_f, W_p, b_p, W_o1, b_o1, W_o2, b_o2):
  L, R = _tc_pre(left_features, right_features, W_l, b_l, W_r)
  lidx = edge_indices[0]
  ridx = edge_indices[1]
  w = edge_features[:, 0]
  we = W_e[0]
  acc_parts, cnt_parts = _sc_edge(L, R, lidx, ridx, w, we)
  out = _tc_post(acc_parts[0], acc_parts[1], cnt_parts.T,
                 right_features, W_f, b_f, W_p, b_p,
                 W_o1[:EMB], W_o1[EMB:], b_o1, W_o2, b_o2)
  return out


# speedup vs baseline: 13.7715x; 13.7715x over previous
import functools

import jax
import jax.numpy as jnp
from jax import lax
from jax.experimental import pallas as pl
from jax.experimental.pallas import tpu as pltpu
from jax.experimental.pallas import tpu_sc as plsc

EMB = 16
N_NODES = 100000
N_EDGES = 3200000
NC = 2
NS = 16
NW = NC * NS
CH = 128
N_CHUNKS = N_EDGES // CH
BASE_CHUNKS = N_CHUNKS // NW
EXTRA = N_CHUNKS - BASE_CHUNKS * NW

ZROW = 1000
ZNCH = N_NODES // ZROW
CNT_N = 102400
CNT_CH = 1024
CNT_NCH = CNT_N // CNT_CH


def _sc_edge_kernel(L_hbm, R_hbm, lidx_hbm, ridx_hbm, w_hbm, we_hbm,
                    acc_out, cnt_out,
                    lidx_v, ridx_v, wch_v, lrows, rrows, mrows,
                    we_v, ones_v, zrow_v, zrows, acc_sh, cnt_sh, sem1, sem2):
  cid = lax.axis_index("c")
  sid = lax.axis_index("s")
  wid = sid * NC + cid

  zero16 = jnp.zeros((16,), jnp.float32)

  def _z_zr(i, _):
    zrows[i] = zero16
    return 0
  lax.fori_loop(0, ZROW, _z_zr, 0)

  def _z_zrow(i, _):
    zrow_v[pl.ds(i * 16, 16)] = zero16
    return 0
  lax.fori_loop(0, CNT_CH // 16, _z_zrow, 0)

  def _o_ones(i, _):
    ones_v[pl.ds(i * 16, 16)] = jnp.ones((16,), jnp.float32)
    return 0
  lax.fori_loop(0, CH // 16, _o_ones, 0)

  pltpu.sync_copy(we_hbm, we_v)

  def _z_acc(k2, _):
    k = sid + k2 * NS

    @pl.when(k < ZNCH)
    def _():
      pltpu.sync_copy(zrows, acc_sh.at[pl.ds(k * ZROW, ZROW)])
    return 0
  lax.fori_loop(0, (ZNCH + NS - 1) // NS, _z_acc, 0)

  def _z_cnt(k2, _):
    k = sid + k2 * NS

    @pl.when(k < CNT_NCH)
    def _():
      pltpu.sync_copy(zrow_v.at[pl.ds(0, CNT_CH)],
                      cnt_sh.at[pl.ds(k * CNT_CH, CNT_CH)])
    return 0
  lax.fori_loop(0, (CNT_NCH + NS - 1) // NS, _z_cnt, 0)

  plsc.subcore_barrier()

  base = wid * BASE_CHUNKS + jnp.minimum(wid, EXTRA)
  count = BASE_CHUNKS + jnp.where(wid < EXTRA, 1, 0)

  def _chunk(k, _):
    off = (base + k) * CH
    pltpu.sync_copy(lidx_hbm.at[pl.ds(off, CH)], lidx_v)
    pltpu.sync_copy(ridx_hbm.at[pl.ds(off, CH)], ridx_v)
    pltpu.sync_copy(w_hbm.at[pl.ds(off, CH)], wch_v)
    cl = pltpu.async_copy(L_hbm.at[lidx_v], lrows, sem1)
    cr = pltpu.async_copy(R_hbm.at[ridx_v], rrows, sem2)
    cl.wait()
    cr.wait()

    we = we_v[...]

    def _group(g, _):
      wg = wch_v[pl.ds(g * 16, 16)]
      base_i = g * 16
      for j in range(16):
        i = base_i + j
        wv = jnp.full((16,), wg[j])
        m = jnp.maximum(lrows[i] + rrows[i] + wv * we, 0.0)
        mrows[i] = m
      return 0
    lax.fori_loop(0, CH // 16, _group, 0)

    pltpu.sync_copy(mrows, acc_sh.at[ridx_v], add=True)
    pltpu.sync_copy(ones_v, cnt_sh.at[ridx_v], add=True)
    return 0
  lax.fori_loop(0, count, _chunk, 0)

  plsc.subcore_barrier()

  def _cp_acc(k2, _):
    k = sid + k2 * NS

    @pl.when(k < ZNCH)
    def _():
      r0 = k * ZROW
      pltpu.sync_copy(acc_sh.at[pl.ds(r0, ZROW)],
                      acc_out.at[cid, pl.ds(r0, ZROW)])
    return 0
  lax.fori_loop(0, (ZNCH + NS - 1) // NS, _cp_acc, 0)

  def _cp_cnt(k2, _):
    k = sid + k2 * NS

    @pl.when(k < CNT_NCH)
    def _():
      pltpu.sync_copy(cnt_sh.at[pl.ds(k * CNT_CH, CNT_CH)],
                      cnt_out.at[cid, pl.ds(k * CNT_CH, CNT_CH)])
    return 0
  lax.fori_loop(0, (CNT_NCH + NS - 1) // NS, _cp_cnt, 0)


def _sc_edge(L, R, lidx, ridx, w, we):
  mesh = plsc.VectorSubcoreMesh(core_axis_name="c", subcore_axis_name="s")
  f = pl.kernel(
      _sc_edge_kernel,
      out_type=(
          jax.ShapeDtypeStruct((NC, N_NODES, EMB), jnp.float32),
          jax.ShapeDtypeStruct((NC, CNT_N), jnp.float32),
      ),
      mesh=mesh,
      scratch_types=[
          pltpu.VMEM((CH,), jnp.int32),
          pltpu.VMEM((CH,), jnp.int32),
          pltpu.VMEM((CH,), jnp.float32),
          pltpu.VMEM((CH, EMB), jnp.float32),
          pltpu.VMEM((CH, EMB), jnp.float32),
          pltpu.VMEM((CH, EMB), jnp.float32),
          pltpu.VMEM((EMB,), jnp.float32),
          pltpu.VMEM((CH,), jnp.float32),
          pltpu.VMEM((CNT_CH,), jnp.float32),
          pltpu.VMEM((ZROW, EMB), jnp.float32),
          pltpu.VMEM_SHARED((N_NODES, EMB), jnp.float32),
          pltpu.VMEM_SHARED((CNT_N,), jnp.float32),
          pltpu.SemaphoreType.DMA,
          pltpu.SemaphoreType.DMA,
      ],
      compiler_params=pltpu.CompilerParams(use_tc_tiling_on_sc=False),
  )
  return f(L, R, lidx, ridx, w, we)


_BLK = 2000
_GRID = N_NODES // _BLK


def _pre_kernel(lf_ref, rf_ref, wl_ref, bl_ref, wr_ref, L_ref, R_ref):
  L_ref[...] = jnp.dot(lf_ref[...], wl_ref[...],
                       preferred_element_type=jnp.float32) + bl_ref[...]
  R_ref[...] = jnp.dot(rf_ref[...], wr_ref[...],
                       preferred_element_type=jnp.float32)


def _tc_pre(lf, rf, W_l, b_l, W_r):
  row_spec = pl.BlockSpec((_BLK, EMB), lambda i: (i, 0))
  w_spec = pl.BlockSpec((EMB, EMB), lambda i: (0, 0))
  b_spec = pl.BlockSpec((1, EMB), lambda i: (0, 0))
  return pl.pallas_call(
      _pre_kernel,
      grid=(_GRID,),
      in_specs=[row_spec, row_spec, w_spec, b_spec, w_spec],
      out_specs=[row_spec, row_spec],
      out_shape=[
          jax.ShapeDtypeStruct((N_NODES, EMB), jnp.float32),
          jax.ShapeDtypeStruct((N_NODES, EMB), jnp.float32),
      ],
  )(lf, rf, W_l, b_l.reshape(1, EMB), W_r)


def _post_kernel(acc0_ref, acc1_ref, cnt_ref, rf_ref, wf_ref, bf_ref,
                 wp_ref, bp_ref, wo1a_ref, wo1b_ref, bo1_ref, wo2_ref,
                 bo2_ref, out_ref):
  acc = acc0_ref[...] + acc1_ref[...]
  n = jnp.sum(cnt_ref[...], axis=1, keepdims=True)
  agg = jnp.dot(acc, wf_ref[...],
                preferred_element_type=jnp.float32) + n * bf_ref[...]
  post = jnp.dot(jnp.maximum(agg, 0.0), wp_ref[...],
                 preferred_element_type=jnp.float32) + bp_ref[...]
  h = jnp.maximum(
      jnp.dot(post, wo1a_ref[...], preferred_element_type=jnp.float32)
      + jnp.dot(rf_ref[...], wo1b_ref[...],
                preferred_element_type=jnp.float32)
      + bo1_ref[...], 0.0)
  out_ref[...] = jnp.dot(h, wo2_ref[...],
                         preferred_element_type=jnp.float32) + bo2_ref[...]


def _tc_post(acc0, acc1, cntT, rf, W_f, b_f, W_p, b_p, W_o1a, W_o1b, b_o1,
             W_o2, b_o2):
  row_spec = pl.BlockSpec((_BLK, EMB), lambda i: (i, 0))
  cnt_spec = pl.BlockSpec((_BLK, NC), lambda i: (i, 0))
  w_spec = pl.BlockSpec((EMB, EMB), lambda i: (0, 0))
  b_spec = pl.BlockSpec((1, EMB), lambda i: (0, 0))
  return pl.pallas_call(
      _post_kernel,
      grid=(_GRID,),
      in_specs=[row_spec, row_spec, cnt_spec, row_spec,
                w_spec, b_spec, w_spec, b_spec,
                w_spec, w_spec, b_spec, w_spec, b_spec],
      out_specs=row_spec,
      out_shape=jax.ShapeDtypeStruct((N_NODES, EMB), jnp.float32),
  )(acc0, acc1, cntT, rf, W_f, b_f.reshape(1, EMB), W_p,
    b_p.reshape(1, EMB), W_o1a, W_o1b, b_o1.reshape(1, EMB), W_o2,
    b_o2.reshape(1, EMB))


@jax.jit
def kernel(left_features, edge_indices, edge_features, right_features,
           W_l, b_l, W_e, W_r, W_f, b_f, W_p, b_p, W_o1, b_o1, W_o2, b_o2):
  L, R = _tc_pre(left_features, right_features, W_l, b_l, W_r)
  lidx = edge_indices[0]
  ridx = edge_indices[1]
  w = edge_features[:, 0]
  we = W_e[0]
  acc_parts, cnt_parts = _sc_edge(L, R, lidx, ridx, w, we)
  out = _tc_post(acc_parts[0], acc_parts[1], cnt_parts[:, :N_NODES].T,
                 right_features, W_f, b_f, W_p, b_p,
                 W_o1[:EMB], W_o1[EMB:], b_o1, W_o2, b_o2)
  return out


# speedup vs baseline: 18.7109x; 1.3587x over previous
import functools

import jax
import jax.numpy as jnp
from jax import lax
from jax.experimental import pallas as pl
from jax.experimental.pallas import tpu as pltpu
from jax.experimental.pallas import tpu_sc as plsc

EMB = 16
N_NODES = 100000
N_EDGES = 3200000
NC = 2
NS = 16
NW = NC * NS
CH = 128
N_CHUNKS = N_EDGES // CH
BASE_CHUNKS = N_CHUNKS // NW
EXTRA = N_CHUNKS - BASE_CHUNKS * NW

ZROW = 200
ZNCH = N_NODES // ZROW
CPROW = 1000
CPNCH = N_NODES // CPROW
CNT_N = 102400
CNT_CH = 1024
CNT_NCH = CNT_N // CNT_CH


def _sc_edge_kernel(L_hbm, R_hbm, ei_hbm, w_hbm, we_hbm,
                    acc_out, cnt_out,
                    ei_v, wch_v, lrows, rrows, mrows,
                    we_v, ones_v, zrow_v, zrows, acc_sh, cnt_sh,
                    semI, semL, semR, semS):
  cid = lax.axis_index("c")
  sid = lax.axis_index("s")
  wid = sid * NC + cid

  zero16 = jnp.zeros((16,), jnp.float32)

  def _z_zr(i, _):
    zrows[i] = zero16
    return 0
  lax.fori_loop(0, ZROW, _z_zr, 0)

  def _z_zrow(i, _):
    zrow_v[pl.ds(i * 16, 16)] = zero16
    return 0
  lax.fori_loop(0, CNT_CH // 16, _z_zrow, 0)

  def _o_ones(i, _):
    ones_v[pl.ds(i * 16, 16)] = jnp.ones((16,), jnp.float32)
    return 0
  lax.fori_loop(0, CH // 16, _o_ones, 0)

  pltpu.sync_copy(we_hbm, we_v)

  def _z_acc(k2, _):
    k = sid + k2 * NS

    @pl.when(k < ZNCH)
    def _():
      pltpu.sync_copy(zrows, acc_sh.at[pl.ds(k * ZROW, ZROW)])
    return 0
  lax.fori_loop(0, (ZNCH + NS - 1) // NS, _z_acc, 0)

  def _z_cnt(k2, _):
    k = sid + k2 * NS

    @pl.when(k < CNT_NCH)
    def _():
      pltpu.sync_copy(zrow_v.at[pl.ds(0, CNT_CH)],
                      cnt_sh.at[pl.ds(k * CNT_CH, CNT_CH)])
    return 0
  lax.fori_loop(0, (CNT_NCH + NS - 1) // NS, _z_cnt, 0)

  plsc.subcore_barrier()

  base = wid * BASE_CHUNKS + jnp.minimum(wid, EXTRA)
  count = BASE_CHUNKS + jnp.where(wid < EXTRA, 1, 0)

  def _issue_idx(k, b):
    off = (base + k) * CH
    pltpu.async_copy(ei_hbm.at[:, pl.ds(off, CH)], ei_v.at[b], semI)
    pltpu.async_copy(w_hbm.at[pl.ds(off, CH)], wch_v.at[b], semI)

  def _wait_idx(k, b):
    off = (base + k) * CH
    pltpu.make_async_copy(ei_hbm.at[:, pl.ds(off, CH)], ei_v.at[b],
                          semI).wait()
    pltpu.make_async_copy(w_hbm.at[pl.ds(off, CH)], wch_v.at[b],
                          semI).wait()

  def _issue_gather(b):
    pltpu.async_copy(L_hbm.at[ei_v.at[b, 0]], lrows.at[b], semL)
    pltpu.async_copy(R_hbm.at[ei_v.at[b, 1]], rrows.at[b], semR)

  def _wait_gather(b):
    pltpu.make_async_copy(L_hbm.at[ei_v.at[b, 0]], lrows.at[b],
                          semL).wait()
    pltpu.make_async_copy(R_hbm.at[ei_v.at[b, 1]], rrows.at[b],
                          semR).wait()

  def _issue_scatter(b):
    pltpu.async_copy(mrows.at[b], acc_sh.at[ei_v.at[b, 1]], semS, add=True)
    pltpu.async_copy(ones_v, cnt_sh.at[ei_v.at[b, 1]], semS, add=True)

  def _wait_scatter(b):
    pltpu.make_async_copy(mrows.at[b], acc_sh.at[ei_v.at[b, 1]],
                          semS).wait()
    pltpu.make_async_copy(ones_v, cnt_sh.at[ei_v.at[b, 1]], semS).wait()

  def _compute(b):
    we = we_v[...]

    def _group(g, _):
      wg = wch_v[b, pl.ds(g * 16, 16)]
      base_i = g * 16
      for j in range(16):
        i = base_i + j
        wv = jnp.full((16,), wg[j])
        m = jnp.maximum(lrows[b, i] + rrows[b, i] + wv * we, 0.0)
        mrows[b, i] = m
      return 0
    lax.fori_loop(0, CH // 16, _group, 0)

  _issue_idx(0, 0)
  _wait_idx(0, 0)
  _issue_gather(0)

  def _chunk(k, _):
    b = lax.rem(k, 2)
    nb = 1 - b

    @pl.when(k > 0)
    def _():
      _wait_scatter(nb)

    @pl.when(k + 1 < count)
    def _():
      _issue_idx(k + 1, nb)

    _wait_gather(b)
    _compute(b)
    _issue_scatter(b)

    @pl.when(k + 1 < count)
    def _():
      _wait_idx(k + 1, nb)
      _issue_gather(nb)
    return 0
  lax.fori_loop(0, count, _chunk, 0)

  _wait_scatter(lax.rem(count - 1, 2))

  plsc.subcore_barrier()

  def _cp_acc(k2, _):
    k = sid + k2 * NS

    @pl.when(k < CPNCH)
    def _():
      r0 = k * CPROW
      pltpu.sync_copy(acc_sh.at[pl.ds(r0, CPROW)],
                      acc_out.at[cid, pl.ds(r0, CPROW)])
    return 0
  lax.fori_loop(0, (CPNCH + NS - 1) // NS, _cp_acc, 0)

  def _cp_cnt(k2, _):
    k = sid + k2 * NS

    @pl.when(k < CNT_NCH)
    def _():
      pltpu.sync_copy(cnt_sh.at[pl.ds(k * CNT_CH, CNT_CH)],
                      cnt_out.at[cid, pl.ds(k * CNT_CH, CNT_CH)])
    return 0
  lax.fori_loop(0, (CNT_NCH + NS - 1) // NS, _cp_cnt, 0)


def _sc_edge(L, R, ei, w, we):
  mesh = plsc.VectorSubcoreMesh(core_axis_name="c", subcore_axis_name="s")
  f = pl.kernel(
      _sc_edge_kernel,
      out_type=(
          jax.ShapeDtypeStruct((NC, N_NODES, EMB), jnp.float32),
          jax.ShapeDtypeStruct((NC, CNT_N), jnp.float32),
      ),
      mesh=mesh,
      scratch_types=[
          pltpu.VMEM((2, 2, CH), jnp.int32),
          pltpu.VMEM((2, CH), jnp.float32),
          pltpu.VMEM((2, CH, EMB), jnp.float32),
          pltpu.VMEM((2, CH, EMB), jnp.float32),
          pltpu.VMEM((2, CH, EMB), jnp.float32),
          pltpu.VMEM((EMB,), jnp.float32),
          pltpu.VMEM((CH,), jnp.float32),
          pltpu.VMEM((CNT_CH,), jnp.float32),
          pltpu.VMEM((ZROW, EMB), jnp.float32),
          pltpu.VMEM_SHARED((N_NODES, EMB), jnp.float32),
          pltpu.VMEM_SHARED((CNT_N,), jnp.float32),
          pltpu.SemaphoreType.DMA,
          pltpu.SemaphoreType.DMA,
          pltpu.SemaphoreType.DMA,
          pltpu.SemaphoreType.DMA,
      ],
      compiler_params=pltpu.CompilerParams(use_tc_tiling_on_sc=False),
  )
  return f(L, R, ei, w, we)


_BLK = 2000
_GRID = N_NODES // _BLK


def _pre_kernel(lf_ref, rf_ref, wl_ref, bl_ref, wr_ref, L_ref, R_ref):
  L_ref[...] = jnp.dot(lf_ref[...], wl_ref[...],
                       preferred_element_type=jnp.float32) + bl_ref[...]
  R_ref[...] = jnp.dot(rf_ref[...], wr_ref[...],
                       preferred_element_type=jnp.float32)


def _tc_pre(lf, rf, W_l, b_l, W_r):
  row_spec = pl.BlockSpec((_BLK, EMB), lambda i: (i, 0))
  w_spec = pl.BlockSpec((EMB, EMB), lambda i: (0, 0))
  b_spec = pl.BlockSpec((1, EMB), lambda i: (0, 0))
  return pl.pallas_call(
      _pre_kernel,
      grid=(_GRID,),
      in_specs=[row_spec, row_spec, w_spec, b_spec, w_spec],
      out_specs=[row_spec, row_spec],
      out_shape=[
          jax.ShapeDtypeStruct((N_NODES, EMB), jnp.float32),
          jax.ShapeDtypeStruct((N_NODES, EMB), jnp.float32),
      ],
  )(lf, rf, W_l, b_l.reshape(1, EMB), W_r)


def _post_kernel(acc0_ref, acc1_ref, cnt_ref, rf_ref, wf_ref, bf_ref,
                 wp_ref, bp_ref, wo1a_ref, wo1b_ref, bo1_ref, wo2_ref,
                 bo2_ref, out_ref):
  acc = acc0_ref[...] + acc1_ref[...]
  n = jnp.sum(cnt_ref[...], axis=1, keepdims=True)
  agg = jnp.dot(acc, wf_ref[...],
                preferred_element_type=jnp.float32) + n * bf_ref[...]
  post = jnp.dot(jnp.maximum(agg, 0.0), wp_ref[...],
                 preferred_element_type=jnp.float32) + bp_ref[...]
  h = jnp.maximum(
      jnp.dot(post, wo1a_ref[...], preferred_element_type=jnp.float32)
      + jnp.dot(rf_ref[...], wo1b_ref[...],
                preferred_element_type=jnp.float32)
      + bo1_ref[...], 0.0)
  out_ref[...] = jnp.dot(h, wo2_ref[...],
                         preferred_element_type=jnp.float32) + bo2_ref[...]


def _tc_post(acc0, acc1, cntT, rf, W_f, b_f, W_p, b_p, W_o1a, W_o1b, b_o1,
             W_o2, b_o2):
  row_spec = pl.BlockSpec((_BLK, EMB), lambda i: (i, 0))
  cnt_spec = pl.BlockSpec((_BLK, NC), lambda i: (i, 0))
  w_spec = pl.BlockSpec((EMB, EMB), lambda i: (0, 0))
  b_spec = pl.BlockSpec((1, EMB), lambda i: (0, 0))
  return pl.pallas_call(
      _post_kernel,
      grid=(_GRID,),
      in_specs=[row_spec, row_spec, cnt_spec, row_spec,
                w_spec, b_spec, w_spec, b_spec,
                w_spec, w_spec, b_spec, w_spec, b_spec],
      out_specs=row_spec,
      out_shape=jax.ShapeDtypeStruct((N_NODES, EMB), jnp.float32),
  )(acc0, acc1, cntT, rf, W_f, b_f.reshape(1, EMB), W_p,
    b_p.reshape(1, EMB), W_o1a, W_o1b, b_o1.reshape(1, EMB), W_o2,
    b_o2.reshape(1, EMB))


@jax.jit
def kernel(left_features, edge_indices, edge_features, right_features,
           W_l, b_l, W_e, W_r, W_f, b_f, W_p, b_p, W_o1, b_o1, W_o2, b_o2):
  L, R = _tc_pre(left_features, right_features, W_l, b_l, W_r)
  w = edge_features[:, 0]
  we = W_e[0]
  acc_parts, cnt_parts = _sc_edge(L, R, edge_indices, w, we)
  out = _tc_post(acc_parts[0], acc_parts[1], cnt_parts[:, :N_NODES].T,
                 right_features, W_f, b_f, W_p, b_p,
                 W_o1[:EMB], W_o1[EMB:], b_o1, W_o2, b_o2)
  return out
